# baseline (speedup 1.0000x reference)
import jax
import jax.numpy as jnp
from jax.experimental import pallas as pl

N = 8192
G = 2048
D = 2
C = 3
K = 16
EPS = 1e-06

BN = 256


def _render_block(x_ref, mus_ref, covs_ref, cols_ref, expand_ref, chan_ref,
                  out_ref):
    x = x_ref[...]
    mu = mus_ref[...]
    cv = covs_ref[...]
    colsp = cols_ref[...]

    x0 = x[:, 0:1]
    x1 = x[:, 1:2]
    dx = x0 - mu[0:1, :]
    dy = x1 - mu[1:2, :]

    c00 = cv[0:1, :]
    c01 = cv[1:2, :]
    c10 = cv[2:3, :]
    c11 = cv[3:4, :]
    inv_det = 1.0 / (c00 * c11 - c01 * c10)
    quad = (c11 * dx * dx - (c01 + c10) * dx * dy + c00 * dy * dy) * inv_det

    q = quad
    vs = []
    mms = []
    for _ in range(K):
        v = jnp.min(q, axis=1, keepdims=True)
        eq = q == v
        eqf = eq.astype(jnp.float32)
        mms.append(jnp.dot(eqf, colsp, preferred_element_type=jnp.float32))
        q = jnp.where(eq, jnp.inf, q)
        vs.append(v)

    V = jnp.concatenate(vs, axis=1)
    EV = jnp.exp(-0.5 * V)
    MM = jnp.concatenate(mms, axis=1)
    CNT = jnp.concatenate([m[:, C:C + 1] for m in mms], axis=1)
    P = CNT
    s = 1
    while s < K:
        P = P + jnp.concatenate(
            [jnp.zeros((BN, s), jnp.float32), P[:, :K - s]], axis=1)
        s *= 2
    allowed = jnp.clip(float(K) - (P - CNT), 0.0, CNT)
    den = jnp.sum(allowed * EV, axis=1, keepdims=True) + EPS
    scale = (allowed / CNT) * EV
    scale_x = jnp.dot(scale, expand_ref[...],
                      preferred_element_type=jnp.float32)
    num = jnp.dot(scale_x * MM, chan_ref[...],
                  preferred_element_type=jnp.float32)
    out_ref[...] = num / den


@jax.jit
def kernel(x, mus, covs, cols):
    mus_t = mus[0].T
    covs4 = covs[0].reshape(G, 4).T
    colsp = jnp.concatenate(
        [cols[0], jnp.ones((G, 1), jnp.float32)], axis=1)
    ridx = jnp.arange(K * (C + 1)) // (C + 1)
    expand = (ridx[None, :] == jnp.arange(K)[:, None]).astype(jnp.float32)
    cidx = jnp.arange(K * (C + 1)) % (C + 1)
    chan = (cidx[:, None] == jnp.arange(C)[None, :]).astype(jnp.float32)
    grid = (N // BN,)
    out = pl.pallas_call(
        _render_block,
        grid=grid,
        in_specs=[
            pl.BlockSpec((BN, D), lambda i: (i, 0)),
            pl.BlockSpec((D, G), lambda i: (0, 0)),
            pl.BlockSpec((4, G), lambda i: (0, 0)),
            pl.BlockSpec((G, C + 1), lambda i: (0, 0)),
            pl.BlockSpec((K, K * (C + 1)), lambda i: (0, 0)),
            pl.BlockSpec((K * (C + 1), C), lambda i: (0, 0)),
        ],
        out_specs=pl.BlockSpec((BN, C), lambda i: (i, 0)),
        out_shape=jax.ShapeDtypeStruct((N, C), jnp.float32),
    )(x, mus_t, covs4, colsp, expand, chan)
    return out


# speedup vs baseline: 1.0373x; 1.0373x over previous
import jax
import jax.numpy as jnp
from jax.experimental import pallas as pl

N = 8192
G = 2048
D = 2
C = 3
K = 16
EPS = 1e-06

BN = 256


def _render_block(x_ref, mus_ref, covs_ref, cols_ref, out_ref):
    x = x_ref[...]
    mu = mus_ref[...]
    cv = covs_ref[...]
    colsp = cols_ref[...]

    x0 = x[:, 0:1]
    x1 = x[:, 1:2]
    dx = x0 - mu[0:1, :]
    dy = x1 - mu[1:2, :]

    c00 = cv[0:1, :]
    c01 = cv[1:2, :]
    c10 = cv[2:3, :]
    c11 = cv[3:4, :]
    inv_det = 1.0 / (c00 * c11 - c01 * c10)
    quad = (c11 * dx * dx - (c01 + c10) * dx * dy + c00 * dy * dy) * inv_det

    q = quad
    vs = []
    mms = []
    for _ in range(K):
        v = jnp.min(q, axis=1, keepdims=True)
        eq = q == v
        eqf = eq.astype(jnp.float32)
        mms.append(jnp.dot(eqf, colsp, preferred_element_type=jnp.float32))
        q = jnp.where(eq, jnp.inf, q)
        vs.append(v)

    V = jnp.concatenate(vs, axis=1)
    EV = jnp.exp(-0.5 * V)
    den = jnp.full((BN, 1), EPS, jnp.float32)
    num = jnp.zeros((BN, C), jnp.float32)
    used = jnp.zeros((BN, 1), jnp.float32)
    for r in range(K):
        cnt = mms[r][:, C:C + 1]
        allowed = jnp.minimum(cnt, float(K) - used)
        used = used + allowed
        ev = EV[:, r:r + 1]
        den = den + allowed * ev
        num = num + ((allowed / cnt) * ev) * mms[r][:, 0:C]
    out_ref[...] = num / den


@jax.jit
def kernel(x, mus, covs, cols):
    mus_t = mus[0].T
    covs4 = covs[0].reshape(G, 4).T
    colsp = jnp.concatenate(
        [cols[0], jnp.ones((G, 1), jnp.float32)], axis=1)
    grid = (N // BN,)
    out = pl.pallas_call(
        _render_block,
        grid=grid,
        in_specs=[
            pl.BlockSpec((BN, D), lambda i: (i, 0)),
            pl.BlockSpec((D, G), lambda i: (0, 0)),
            pl.BlockSpec((4, G), lambda i: (0, 0)),
            pl.BlockSpec((G, C + 1), lambda i: (0, 0)),
        ],
        out_specs=pl.BlockSpec((BN, C), lambda i: (i, 0)),
        out_shape=jax.ShapeDtypeStruct((N, C), jnp.float32),
    )(x, mus_t, covs4, colsp)
    return out
